# Initial kernel scaffold; baseline (speedup 1.0000x reference)
import functools

import jax
import jax.numpy as jnp
from jax import lax
from jax.experimental import pallas as pl
from jax.experimental.pallas import tpu as pltpu
from jax.experimental.pallas import tpu_sc as plsc

N = 100000
E = 1600000
G = 256
D = 16
H = 32

F32 = jnp.float32


def _stats_body(x_ref, bnw_ref, bnb_ref, out_ref):
    x = x_ref[...]
    mean = jnp.sum(x, axis=0, keepdims=True) / N
    mean2 = jnp.sum(x * x, axis=0, keepdims=True) / N
    var = mean2 - mean * mean
    scale = bnw_ref[...] / jnp.sqrt(var + 1e-5)
    shift = bnb_ref[...] - mean * scale
    out_ref[0:1, :] = scale
    out_ref[1:2, :] = shift
    out_ref[2:8, :] = jnp.zeros((6, D), F32)


_stats_call = pl.pallas_call(
    _stats_body,
    out_shape=jax.ShapeDtypeStruct((8, D), F32),
)


_NBLK = 4000
_NGRID = N // _NBLK


def _node_body(x_ref, st_ref, b_ref, w1_ref, b1_ref, w2_ref, b2_ref,
               xc_ref, xs_ref):
    i = pl.program_id(0)
    x = x_ref[...]
    xn = x * st_ref[0:1, :] + st_ref[1:2, :]
    h1 = jnp.maximum(
        jnp.dot(xn, w1_ref[...], precision=lax.Precision.HIGHEST)
        + b1_ref[...], 0.0)
    hn = jnp.tanh(
        jnp.dot(h1, w2_ref[...], precision=lax.Precision.HIGHEST)
        + b2_ref[...])
    xc_ref[...] = jnp.concatenate([hn, xn], axis=1)

    g = b_ref[0, 0, :]
    rows = lax.broadcasted_iota(F32, (G, _NBLK), 0)
    oh = (rows == g[None, :]).astype(F32)
    xa = jnp.concatenate(
        [xn, jnp.ones((_NBLK, 1), F32), jnp.zeros((_NBLK, 15), F32)], axis=1)
    part = jnp.dot(oh, xa, precision=lax.Precision.HIGHEST)

    @pl.when(i == 0)
    def _():
        xs_ref[...] = part

    @pl.when(i > 0)
    def _():
        xs_ref[...] = xs_ref[...] + part


_node_call = pl.pallas_call(
    _node_body,
    grid=(_NGRID,),
    in_specs=[
        pl.BlockSpec((_NBLK, D), lambda i: (i, 0)),
        pl.BlockSpec((8, D), lambda i: (0, 0)),
        pl.BlockSpec((1, 1, _NBLK), lambda i: (i, 0, 0)),
        pl.BlockSpec((D, H), lambda i: (0, 0)),
        pl.BlockSpec((1, H), lambda i: (0, 0)),
        pl.BlockSpec((H, H), lambda i: (0, 0)),
        pl.BlockSpec((1, H), lambda i: (0, 0)),
    ],
    out_specs=[
        pl.BlockSpec((_NBLK, H + D), lambda i: (i, 0)),
        pl.BlockSpec((G, 32), lambda i: (0, 0)),
    ],
    out_shape=[
        jax.ShapeDtypeStruct((N, H + D), F32),
        jax.ShapeDtypeStruct((G, 32), F32),
    ],
)


_NC = 2
_NS = 16
_NW = _NC * _NS
_EPW = E // _NW
_CH = 80
_MAC = 400
_NMAC = _EPW // _MAC


def _gather_body(xc_hbm, dst_hbm, src_hbm, qd_hbm, qs_hbm,
                 dsti, srci, qd_v, qs_v, sem):
    c = lax.axis_index("c")
    s = lax.axis_index("s")
    wid = s * _NC + c
    base = wid * _EPW

    def macro(k, carry):
        off = base + k * _MAC
        pltpu.sync_copy(dst_hbm.at[pl.ds(off, _MAC)], dsti)
        pltpu.sync_copy(src_hbm.at[pl.ds(off, _MAC)], srci)
        copies = []
        for j in range(_MAC // _CH):
            sl = pl.ds(j * _CH, _CH)
            copies.append(pltpu.async_copy(
                xc_hbm.at[dsti.at[sl]], qd_v.at[sl], sem))
            copies.append(pltpu.async_copy(
                xc_hbm.at[srci.at[sl]], qs_v.at[sl], sem))
        for cp in copies:
            cp.wait()
        pltpu.sync_copy(qd_v, qd_hbm.at[pl.ds(off, _MAC)])
        pltpu.sync_copy(qs_v, qs_hbm.at[pl.ds(off, _MAC)])
        return carry

    lax.fori_loop(0, _NMAC, macro, 0)


_gather_call = pl.kernel(
    _gather_body,
    out_type=[
        jax.ShapeDtypeStruct((E, H + D), F32),
        jax.ShapeDtypeStruct((E, H + D), F32),
    ],
    mesh=plsc.VectorSubcoreMesh(
        core_axis_name="c", subcore_axis_name="s",
        num_cores=_NC, num_subcores=_NS),
    scratch_types=[
        pltpu.VMEM((_MAC,), jnp.int32),
        pltpu.VMEM((_MAC,), jnp.int32),
        pltpu.VMEM((_MAC, H + D), F32),
        pltpu.VMEM((_MAC, H + D), F32),
        pltpu.SemaphoreType.DMA,
    ],
)


_EBLK = 3200
_EGRID = E // _EBLK


def _edge_body(qd_ref, qs_ref, d_ref, xs_ref, wt_ref, wb_ref, bc1_ref,
               wc2_ref, bc2_ref, wo1_ref, bo1_ref, wo2_ref, bo2_ref,
               out_ref, acc_ref):
    i = pl.program_id(0)
    pre = (jnp.dot(qd_ref[...], wt_ref[...], precision=lax.Precision.HIGHEST)
           + jnp.dot(qs_ref[...], wb_ref[...], precision=lax.Precision.HIGHEST)
           + bc1_ref[...])
    h = jnp.maximum(pre, 0.0)
    m = jnp.tanh(
        jnp.dot(h, wc2_ref[...], precision=lax.Precision.HIGHEST)
        + bc2_ref[...])

    cnt = xs_ref[:, 16:17]
    gidx = lax.broadcasted_iota(F32, (G, G), 0)
    jidx = lax.broadcasted_iota(F32, (G, G), 1)
    lt = (jidx < gidx).astype(F32)
    starts = jnp.dot(lt, cnt, precision=lax.Precision.HIGHEST)
    ends = starts + cnt

    d = d_ref[0, 0, :]
    oh = ((d[None, :] >= starts) & (d[None, :] < ends)).astype(F32)
    part = jnp.dot(oh, m, precision=lax.Precision.HIGHEST)

    @pl.when(i == 0)
    def _():
        acc_ref[...] = part

    @pl.when(i > 0)
    def _():
        acc_ref[...] = acc_ref[...] + part

    @pl.when(i == _EGRID - 1)
    def _():
        sums = jnp.concatenate([acc_ref[...], xs_ref[:, :D]], axis=1)
        xm = sums / jnp.maximum(cnt, 1.0)
        o1 = jnp.maximum(
            jnp.dot(xm, wo1_ref[...], precision=lax.Precision.HIGHEST)
            + bo1_ref[...], 0.0)
        z = (jnp.dot(o1, wo2_ref[...], precision=lax.Precision.HIGHEST)
             + bo2_ref[...])
        out_ref[...] = 1.0 / (1.0 + jnp.exp(-z))


_edge_call = pl.pallas_call(
    _edge_body,
    grid=(_EGRID,),
    in_specs=[
        pl.BlockSpec((_EBLK, H + D), lambda i: (i, 0)),
        pl.BlockSpec((_EBLK, H + D), lambda i: (i, 0)),
        pl.BlockSpec((1, 1, _EBLK), lambda i: (i, 0, 0)),
        pl.BlockSpec((G, 32), lambda i: (0, 0)),
        pl.BlockSpec((H + D, 2 * H), lambda i: (0, 0)),
        pl.BlockSpec((H + D, 2 * H), lambda i: (0, 0)),
        pl.BlockSpec((1, 2 * H), lambda i: (0, 0)),
        pl.BlockSpec((2 * H, H), lambda i: (0, 0)),
        pl.BlockSpec((1, H), lambda i: (0, 0)),
        pl.BlockSpec((H + D, H), lambda i: (0, 0)),
        pl.BlockSpec((1, H), lambda i: (0, 0)),
        pl.BlockSpec((H, 1), lambda i: (0, 0)),
        pl.BlockSpec((1, 1), lambda i: (0, 0)),
    ],
    out_specs=pl.BlockSpec((G, 1), lambda i: (0, 0)),
    out_shape=jax.ShapeDtypeStruct((G, 1), F32),
    scratch_shapes=[pltpu.VMEM((G, H), F32)],
)


def kernel(x, edge_index, batch, bn_w, bn_b, W1, b1, W2, b2,
           Wc1, bc1, Wc2, bc2, Wo1, bo1, Wo2, bo2):
    src = edge_index[0]
    dst = edge_index[1]

    stats = _stats_call(x, bn_w.reshape(1, D), bn_b.reshape(1, D))

    batch3 = batch.astype(F32).reshape(_NGRID, 1, _NBLK)
    xc, xs = _node_call(x, stats, batch3,
                        W1, b1.reshape(1, H), W2, b2.reshape(1, H))

    qd, qs = _gather_call(xc, dst, src)

    wtop = Wc1[:H + D] - Wc1[H + D:]
    wbot = Wc1[H + D:]
    dst3 = dst.astype(F32).reshape(_EGRID, 1, _EBLK)
    out = _edge_call(qd, qs, dst3, xs, wtop, wbot, bc1.reshape(1, 2 * H),
                     Wc2, bc2.reshape(1, H), Wo1, bo1.reshape(1, H),
                     Wo2, bo2.reshape(1, 1))
    return out


# speedup vs baseline: 2.3390x; 2.3390x over previous
import functools

import jax
import jax.numpy as jnp
from jax import lax
from jax.experimental import pallas as pl
from jax.experimental.pallas import tpu as pltpu
from jax.experimental.pallas import tpu_sc as plsc

N = 100000
E = 1600000
G = 256
D = 16
H = 32

F32 = jnp.float32


def _stats_body(x_ref, bnw_ref, bnb_ref, out_ref):
    x = x_ref[...]
    mean = jnp.sum(x, axis=0, keepdims=True) / N
    mean2 = jnp.sum(x * x, axis=0, keepdims=True) / N
    var = mean2 - mean * mean
    scale = bnw_ref[...] / jnp.sqrt(var + 1e-5)
    shift = bnb_ref[...] - mean * scale
    out_ref[0:1, :] = scale
    out_ref[1:2, :] = shift
    out_ref[2:8, :] = jnp.zeros((6, D), F32)


_stats_call = pl.pallas_call(
    _stats_body,
    out_shape=jax.ShapeDtypeStruct((8, D), F32),
)


_NBLK = 4000
_NGRID = N // _NBLK


def _node_body(x_ref, st_ref, b_ref, w1_ref, b1_ref, w2_ref, b2_ref,
               wt_ref, wb_ref, bc1_ref, t_ref, xs_ref):
    i = pl.program_id(0)
    x = x_ref[...]
    xn = x * st_ref[0:1, :] + st_ref[1:2, :]
    h1 = jnp.maximum(
        jnp.dot(xn, w1_ref[...], precision=lax.Precision.HIGHEST)
        + b1_ref[...], 0.0)
    hn = jnp.tanh(
        jnp.dot(h1, w2_ref[...], precision=lax.Precision.HIGHEST)
        + b2_ref[...])
    xc = jnp.concatenate([hn, xn], axis=1)
    a = (jnp.dot(xc, wt_ref[...], precision=lax.Precision.HIGHEST)
         + bc1_ref[...])
    b = jnp.dot(xc, wb_ref[...], precision=lax.Precision.HIGHEST)
    t_ref[...] = jnp.concatenate([a, b], axis=1)

    g = b_ref[0, 0, :]
    rows = lax.broadcasted_iota(jnp.int32, (G, _NBLK), 0).astype(F32)
    oh = (rows == g[None, :]).astype(F32)
    xa = jnp.concatenate(
        [xn, jnp.ones((_NBLK, 1), F32), jnp.zeros((_NBLK, 15), F32)], axis=1)
    part = jnp.dot(oh, xa, precision=lax.Precision.HIGHEST)

    @pl.when(i == 0)
    def _():
        xs_ref[...] = part

    @pl.when(i > 0)
    def _():
        xs_ref[...] = xs_ref[...] + part


_node_call = pl.pallas_call(
    _node_body,
    grid=(_NGRID,),
    in_specs=[
        pl.BlockSpec((_NBLK, D), lambda i: (i, 0)),
        pl.BlockSpec((8, D), lambda i: (0, 0)),
        pl.BlockSpec((1, 1, _NBLK), lambda i: (i, 0, 0)),
        pl.BlockSpec((D, H), lambda i: (0, 0)),
        pl.BlockSpec((1, H), lambda i: (0, 0)),
        pl.BlockSpec((H, H), lambda i: (0, 0)),
        pl.BlockSpec((1, H), lambda i: (0, 0)),
        pl.BlockSpec((H + D, 2 * H), lambda i: (0, 0)),
        pl.BlockSpec((H + D, 2 * H), lambda i: (0, 0)),
        pl.BlockSpec((1, 2 * H), lambda i: (0, 0)),
    ],
    out_specs=[
        pl.BlockSpec((_NBLK, 128), lambda i: (i, 0)),
        pl.BlockSpec((G, 32), lambda i: (0, 0)),
    ],
    out_shape=[
        jax.ShapeDtypeStruct((N, 128), F32),
        jax.ShapeDtypeStruct((G, 32), F32),
    ],
)


_NC = 2
_NS = 16
_NW = _NC * _NS
_EPW = E // _NW
_CH = 80
_MAC = 400
_NSTR = _MAC // _CH
_NMAC = _EPW // _MAC


def _gather_body(t_hbm, dst_hbm, src_hbm, qd_hbm, qs_hbm,
                 dsti, srci, qd_v, qs_v, sem):
    c = lax.axis_index("c")
    s = lax.axis_index("s")
    wid = s * _NC + c
    base = wid * _EPW

    def macro(k, carry):
        off = base + k * _MAC
        pltpu.sync_copy(dst_hbm.at[pl.ds(off, _MAC)], dsti)
        pltpu.sync_copy(src_hbm.at[pl.ds(off, _MAC)], srci)
        copies = []
        for j in range(_NSTR):
            sl = pl.ds(j * _CH, _CH)
            copies.append(pltpu.async_copy(
                t_hbm.at[dsti.at[sl]], qd_v.at[sl], sem))
            copies.append(pltpu.async_copy(
                t_hbm.at[srci.at[sl]], qs_v.at[sl], sem))
        for cp in copies:
            cp.wait()
        pltpu.sync_copy(qd_v, qd_hbm.at[pl.ds(off, _MAC)])
        pltpu.sync_copy(qs_v, qs_hbm.at[pl.ds(off, _MAC)])
        return carry

    lax.fori_loop(0, _NMAC, macro, 0)


@functools.cache
def _make_gather_call():
    return pl.kernel(
        _gather_body,
        out_type=[
            jax.ShapeDtypeStruct((E, 128), F32),
            jax.ShapeDtypeStruct((E, 128), F32),
        ],
        mesh=plsc.VectorSubcoreMesh(
            core_axis_name="c", subcore_axis_name="s",
            num_cores=_NC, num_subcores=_NS),
        scratch_types=[
            pltpu.VMEM((_MAC,), jnp.int32),
            pltpu.VMEM((_MAC,), jnp.int32),
            pltpu.VMEM((_MAC, 128), F32),
            pltpu.VMEM((_MAC, 128), F32),
            pltpu.SemaphoreType.DMA,
        ],
    )


_EBLK = 3200
_EGRID = E // _EBLK


def _edge_body(qd_ref, qs_ref, d_ref, xs_ref,
               wc2_ref, bc2_ref, wo1_ref, bo1_ref, wo2_ref, bo2_ref,
               out_ref, acc_ref):
    i = pl.program_id(0)
    pre = qd_ref[:, :2 * H] + qs_ref[:, 2 * H:]
    h = jnp.maximum(pre, 0.0)
    m = jnp.tanh(
        jnp.dot(h, wc2_ref[...], precision=lax.Precision.HIGHEST)
        + bc2_ref[...])

    cnt = xs_ref[:, 16:17]
    gidx = lax.broadcasted_iota(jnp.int32, (G, G), 0)
    jidx = lax.broadcasted_iota(jnp.int32, (G, G), 1)
    lt = (jidx < gidx).astype(F32)
    starts = jnp.dot(lt, cnt, precision=lax.Precision.HIGHEST)
    ends = starts + cnt

    d = d_ref[0, 0, :]
    oh = ((d[None, :] >= starts) & (d[None, :] < ends)).astype(F32)
    part = jnp.dot(oh, m, precision=lax.Precision.HIGHEST)

    @pl.when(i == 0)
    def _():
        acc_ref[...] = part

    @pl.when(i > 0)
    def _():
        acc_ref[...] = acc_ref[...] + part

    @pl.when(i == _EGRID - 1)
    def _():
        sums = jnp.concatenate([acc_ref[...], xs_ref[:, :D]], axis=1)
        xm = sums / jnp.maximum(cnt, 1.0)
        o1 = jnp.maximum(
            jnp.dot(xm, wo1_ref[...], precision=lax.Precision.HIGHEST)
            + bo1_ref[...], 0.0)
        z = (jnp.dot(o1, wo2_ref[...], precision=lax.Precision.HIGHEST)
             + bo2_ref[...])
        out_ref[...] = 1.0 / (1.0 + jnp.exp(-z))


_edge_call = pl.pallas_call(
    _edge_body,
    grid=(_EGRID,),
    in_specs=[
        pl.BlockSpec((_EBLK, 128), lambda i: (i, 0)),
        pl.BlockSpec((_EBLK, 128), lambda i: (i, 0)),
        pl.BlockSpec((1, 1, _EBLK), lambda i: (i, 0, 0)),
        pl.BlockSpec((G, 32), lambda i: (0, 0)),
        pl.BlockSpec((2 * H, H), lambda i: (0, 0)),
        pl.BlockSpec((1, H), lambda i: (0, 0)),
        pl.BlockSpec((H + D, H), lambda i: (0, 0)),
        pl.BlockSpec((1, H), lambda i: (0, 0)),
        pl.BlockSpec((H, 1), lambda i: (0, 0)),
        pl.BlockSpec((1, 1), lambda i: (0, 0)),
    ],
    out_specs=pl.BlockSpec((G, 1), lambda i: (0, 0)),
    out_shape=jax.ShapeDtypeStruct((G, 1), F32),
    scratch_shapes=[pltpu.VMEM((G, H), F32)],
)


def kernel(x, edge_index, batch, bn_w, bn_b, W1, b1, W2, b2,
           Wc1, bc1, Wc2, bc2, Wo1, bo1, Wo2, bo2):
    src = edge_index[0]
    dst = edge_index[1]

    stats = _stats_call(x, bn_w.reshape(1, D), bn_b.reshape(1, D))

    wtop = Wc1[:H + D] - Wc1[H + D:]
    wbot = Wc1[H + D:]
    batch3 = batch.astype(F32).reshape(_NGRID, 1, _NBLK)
    t, xs = _node_call(x, stats, batch3,
                       W1, b1.reshape(1, H), W2, b2.reshape(1, H),
                       wtop, wbot, bc1.reshape(1, 2 * H))

    qd, qs = _make_gather_call()(t, dst, src)

    dst3 = dst.astype(F32).reshape(_EGRID, 1, _EBLK)
    out = _edge_call(qd, qs, dst3, xs,
                     Wc2, bc2.reshape(1, H), Wo1, bo1.reshape(1, H),
                     Wo2, bo2.reshape(1, 1))
    return out


# speedup vs baseline: 5.8324x; 2.4935x over previous
import functools

import jax
import jax.numpy as jnp
from jax import lax
from jax.experimental import pallas as pl
from jax.experimental.pallas import tpu as pltpu
from jax.experimental.pallas import tpu_sc as plsc

N = 100000
E = 1600000
G = 256
D = 16
H = 32

F32 = jnp.float32
BF16 = jnp.bfloat16


def _stats_body(x_ref, bnw_ref, bnb_ref, out_ref):
    x = x_ref[...]
    mean = jnp.sum(x, axis=0, keepdims=True) / N
    mean2 = jnp.sum(x * x, axis=0, keepdims=True) / N
    var = mean2 - mean * mean
    scale = bnw_ref[...] / jnp.sqrt(var + 1e-5)
    shift = bnb_ref[...] - mean * scale
    out_ref[0:1, :] = scale
    out_ref[1:2, :] = shift
    out_ref[2:8, :] = jnp.zeros((6, D), F32)


_stats_call = pl.pallas_call(
    _stats_body,
    out_shape=jax.ShapeDtypeStruct((8, D), F32),
)


_NBLK = 4000
_NGRID = N // _NBLK


def _node_body(x_ref, st_ref, b_ref, w1_ref, b1_ref, w2_ref, b2_ref,
               wh_ref, wx_ref, bt_ref, t_ref, xs_ref):
    i = pl.program_id(0)
    x = x_ref[...]
    xn = x * st_ref[0:1, :] + st_ref[1:2, :]
    h1 = jnp.maximum(
        jnp.dot(xn, w1_ref[...], precision=lax.Precision.HIGHEST)
        + b1_ref[...], 0.0)
    hn = jnp.tanh(
        jnp.dot(h1, w2_ref[...], precision=lax.Precision.HIGHEST)
        + b2_ref[...])
    t_ref[...] = (
        jnp.dot(hn, wh_ref[...], precision=lax.Precision.HIGHEST)
        + jnp.dot(xn, wx_ref[...], precision=lax.Precision.HIGHEST)
        + bt_ref[...])

    g = b_ref[0, 0, :]
    rows = lax.broadcasted_iota(jnp.int32, (G, _NBLK), 0).astype(F32)
    oh = (rows == g[None, :]).astype(BF16)
    xa = jnp.concatenate(
        [xn, jnp.ones((_NBLK, 1), F32), jnp.zeros((_NBLK, 15), F32)],
        axis=1).astype(BF16)
    part = jnp.dot(oh, xa, preferred_element_type=F32)

    @pl.when(i == 0)
    def _():
        xs_ref[...] = part

    @pl.when(i > 0)
    def _():
        xs_ref[...] = xs_ref[...] + part

    @pl.when(i == _NGRID - 1)
    def _():
        cnt = xs_ref[:, 16:17]
        gidx = lax.broadcasted_iota(jnp.int32, (G, G), 0)
        jidx = lax.broadcasted_iota(jnp.int32, (G, G), 1)
        lt = (jidx < gidx).astype(F32)
        starts = jnp.dot(lt, cnt, precision=lax.Precision.HIGHEST)
        xs_ref[:, 17:18] = starts


_node_call = pl.pallas_call(
    _node_body,
    grid=(_NGRID,),
    in_specs=[
        pl.BlockSpec((_NBLK, D), lambda i: (i, 0)),
        pl.BlockSpec((8, D), lambda i: (0, 0)),
        pl.BlockSpec((1, 1, _NBLK), lambda i: (i, 0, 0)),
        pl.BlockSpec((D, H), lambda i: (0, 0)),
        pl.BlockSpec((1, H), lambda i: (0, 0)),
        pl.BlockSpec((H, H), lambda i: (0, 0)),
        pl.BlockSpec((1, H), lambda i: (0, 0)),
        pl.BlockSpec((H, 128), lambda i: (0, 0)),
        pl.BlockSpec((D, 128), lambda i: (0, 0)),
        pl.BlockSpec((1, 128), lambda i: (0, 0)),
    ],
    out_specs=[
        pl.BlockSpec((_NBLK, 128), lambda i: (i, 0)),
        pl.BlockSpec((G, 32), lambda i: (0, 0)),
    ],
    out_shape=[
        jax.ShapeDtypeStruct((N, 128), F32),
        jax.ShapeDtypeStruct((G, 32), F32),
    ],
)


_NC = 2
_NS = 16
_NW = _NC * _NS
_MAC = 200
_CHS = ((0, 104), (104, 96))


def _gather_body(nmac, epw, t_hbm, dst_hbm, src_hbm, qd_hbm, qs_hbm,
                 dsti_a, srci_a, qd_a, qs_a,
                 dsti_b, srci_b, qd_b, qs_b, sem_a, sem_b):
    c = lax.axis_index("c")
    s = lax.axis_index("s")
    wid = s * _NC + c
    base = wid * epw

    def start(off, dsti, srci, qd_v, qs_v, sem):
        pltpu.sync_copy(dst_hbm.at[pl.ds(off, _MAC)], dsti)
        pltpu.sync_copy(src_hbm.at[pl.ds(off, _MAC)], srci)
        for o, n in _CHS:
            sl = pl.ds(o, n)
            pltpu.async_copy(t_hbm.at[dsti.at[sl]], qd_v.at[sl], sem)
            pltpu.async_copy(t_hbm.at[srci.at[sl]], qs_v.at[sl], sem)

    def finish(off, dsti, srci, qd_v, qs_v, sem):
        for o, n in _CHS:
            sl = pl.ds(o, n)
            pltpu.make_async_copy(t_hbm.at[dsti.at[sl]], qd_v.at[sl],
                                  sem).wait()
            pltpu.make_async_copy(t_hbm.at[srci.at[sl]], qs_v.at[sl],
                                  sem).wait()
        pltpu.sync_copy(qd_v, qd_hbm.at[pl.ds(off, _MAC)])
        pltpu.sync_copy(qs_v, qs_hbm.at[pl.ds(off, _MAC)])

    bufs_a = (dsti_a, srci_a, qd_a, qs_a, sem_a)
    bufs_b = (dsti_b, srci_b, qd_b, qs_b, sem_b)

    start(base, *bufs_a)

    def body(i, carry):
        k = 2 * i
        start(base + (k + 1) * _MAC, *bufs_b)
        finish(base + k * _MAC, *bufs_a)

        @pl.when(k + 2 < nmac)
        def _():
            start(base + (k + 2) * _MAC, *bufs_a)

        finish(base + (k + 1) * _MAC, *bufs_b)
        return carry

    lax.fori_loop(0, nmac // 2, body, 0)
    if nmac % 2 == 1:
        finish(base + (nmac - 1) * _MAC, *bufs_a)


@functools.cache
def _make_gather_call(ne):
    epw = ne // _NW
    nmac = epw // _MAC
    return pl.kernel(
        functools.partial(_gather_body, nmac, epw),
        out_type=[
            jax.ShapeDtypeStruct((ne, 128), F32),
            jax.ShapeDtypeStruct((ne, 128), F32),
        ],
        mesh=plsc.VectorSubcoreMesh(
            core_axis_name="c", subcore_axis_name="s",
            num_cores=_NC, num_subcores=_NS),
        scratch_types=[
            pltpu.VMEM((_MAC,), jnp.int32),
            pltpu.VMEM((_MAC,), jnp.int32),
            pltpu.VMEM((_MAC, 128), F32),
            pltpu.VMEM((_MAC, 128), F32),
            pltpu.VMEM((_MAC,), jnp.int32),
            pltpu.VMEM((_MAC,), jnp.int32),
            pltpu.VMEM((_MAC, 128), F32),
            pltpu.VMEM((_MAC, 128), F32),
            pltpu.SemaphoreType.DMA,
            pltpu.SemaphoreType.DMA,
        ],
    )


_EBLK = 3200


def _edge_body(egrid, qd_ref, qs_ref, d_ref, xs_ref, wc2_ref, bc2_ref,
               acc_ref):
    i = pl.program_id(0)
    pre = qd_ref[:, :2 * H] + qs_ref[:, 2 * H:]
    h = jnp.maximum(pre, 0.0).astype(BF16)
    m = jnp.tanh(
        jnp.dot(h, wc2_ref[...].astype(BF16), preferred_element_type=F32)
        + bc2_ref[...])

    cnt = xs_ref[:, 16:17]
    starts = xs_ref[:, 17:18]
    ends = starts + cnt

    d = d_ref[0, 0, :]
    oh = ((d[None, :] >= starts)
          & (d[None, :] < ends)).astype(BF16)
    part = jnp.dot(oh, m.astype(BF16),
                   preferred_element_type=F32)

    @pl.when(i == 0)
    def _():
        acc_ref[...] = part

    @pl.when(i > 0)
    def _():
        acc_ref[...] = acc_ref[...] + part


@functools.cache
def _make_edge_call(ne):
    egrid = ne // _EBLK
    return pl.pallas_call(
        functools.partial(_edge_body, egrid),
        grid=(egrid,),
        in_specs=[
            pl.BlockSpec((_EBLK, 128), lambda i: (i, 0)),
            pl.BlockSpec((_EBLK, 128), lambda i: (i, 0)),
            pl.BlockSpec((1, 1, _EBLK), lambda i: (i, 0, 0)),
            pl.BlockSpec((G, 32), lambda i: (0, 0)),
            pl.BlockSpec((2 * H, H), lambda i: (0, 0)),
            pl.BlockSpec((1, H), lambda i: (0, 0)),
        ],
        out_specs=pl.BlockSpec((G, H), lambda i: (0, 0)),
        out_shape=jax.ShapeDtypeStruct((G, H), F32),
    )


def _final_body(acc1_ref, acc2_ref, xs_ref, wo1_ref, bo1_ref,
                wo2_ref, bo2_ref, out_ref):
    cnt = xs_ref[:, 16:17]
    sums = jnp.concatenate(
        [acc1_ref[...] + acc2_ref[...], xs_ref[:, :D]], axis=1)
    xm = sums / jnp.maximum(cnt, 1.0)
    o1 = jnp.maximum(
        jnp.dot(xm, wo1_ref[...], precision=lax.Precision.HIGHEST)
        + bo1_ref[...], 0.0)
    z = (jnp.dot(o1, wo2_ref[...], precision=lax.Precision.HIGHEST)
         + bo2_ref[...])
    out_ref[...] = 1.0 / (1.0 + jnp.exp(-z))


_final_call = pl.pallas_call(
    _final_body,
    out_shape=jax.ShapeDtypeStruct((G, 1), F32),
)


def kernel(x, edge_index, batch, bn_w, bn_b, W1, b1, W2, b2,
           Wc1, bc1, Wc2, bc2, Wo1, bo1, Wo2, bo2):
    src = edge_index[0]
    dst = edge_index[1]
    e2 = E // 2

    stats = _stats_call(x, bn_w.reshape(1, D), bn_b.reshape(1, D))

    w128 = jnp.concatenate([Wc1[:H + D] - Wc1[H + D:], Wc1[H + D:]], axis=1)
    bt = jnp.concatenate([bc1, jnp.zeros((2 * H,), F32)]).reshape(1, 128)
    batch3 = batch.astype(F32).reshape(_NGRID, 1, _NBLK)
    t, xs = _node_call(x, stats, batch3,
                       W1, b1.reshape(1, H), W2, b2.reshape(1, H),
                       w128[:H], w128[H:], bt)

    gather = _make_gather_call(e2)
    edge = _make_edge_call(e2)
    dstf = dst.astype(F32)
    bc2r = bc2.reshape(1, H)

    qd1, qs1 = gather(t, dst[:e2], src[:e2])
    qd2, qs2 = gather(t, dst[e2:], src[e2:])
    d3_1 = dstf[:e2].reshape(e2 // _EBLK, 1, _EBLK)
    d3_2 = dstf[e2:].reshape(e2 // _EBLK, 1, _EBLK)
    acc1 = edge(qd1, qs1, d3_1, xs, Wc2, bc2r)
    acc2 = edge(qd2, qs2, d3_2, xs, Wc2, bc2r)

    out = _final_call(acc1, acc2, xs, Wo1, bo1.reshape(1, H),
                      Wo2, bo2.reshape(1, 1))
    return out


# speedup vs baseline: 8.6982x; 1.4914x over previous
import functools

import jax
import jax.numpy as jnp
from jax import lax
from jax.experimental import pallas as pl
from jax.experimental.pallas import tpu as pltpu
from jax.experimental.pallas import tpu_sc as plsc

N = 100000
E = 1600000
G = 256
D = 16
H = 32

F32 = jnp.float32
BF16 = jnp.bfloat16


def _stats_body(x_ref, bnw_ref, bnb_ref, out_ref):
    x = x_ref[...]
    mean = jnp.sum(x, axis=0, keepdims=True) / N
    mean2 = jnp.sum(x * x, axis=0, keepdims=True) / N
    var = mean2 - mean * mean
    scale = bnw_ref[...] / jnp.sqrt(var + 1e-5)
    shift = bnb_ref[...] - mean * scale
    out_ref[0:1, :] = scale
    out_ref[1:2, :] = shift
    out_ref[2:8, :] = jnp.zeros((6, D), F32)


_stats_call = pl.pallas_call(
    _stats_body,
    out_shape=jax.ShapeDtypeStruct((8, D), F32),
)


_NBLK = 4000
_NGRID = N // _NBLK


def _node_body(x_ref, st_ref, b_ref, w1_ref, b1_ref, w2_ref, b2_ref,
               wh_ref, wx_ref, bt_ref, t_ref, xs_ref):
    i = pl.program_id(0)
    x = x_ref[...]
    xn = x * st_ref[0:1, :] + st_ref[1:2, :]
    h1 = jnp.maximum(
        jnp.dot(xn, w1_ref[...], precision=lax.Precision.HIGHEST)
        + b1_ref[...], 0.0)
    hn = jnp.tanh(
        jnp.dot(h1, w2_ref[...], precision=lax.Precision.HIGHEST)
        + b2_ref[...])
    t_ref[...] = (
        jnp.dot(hn, wh_ref[...], precision=lax.Precision.HIGHEST)
        + jnp.dot(xn, wx_ref[...], precision=lax.Precision.HIGHEST)
        + bt_ref[...])

    g = b_ref[0, 0, :]
    rows = lax.broadcasted_iota(jnp.int32, (G, _NBLK), 0).astype(F32)
    oh = (rows == g[None, :]).astype(BF16)
    xa = jnp.concatenate(
        [xn, jnp.ones((_NBLK, 1), F32), jnp.zeros((_NBLK, 15), F32)],
        axis=1).astype(BF16)
    part = jnp.dot(oh, xa, preferred_element_type=F32)

    @pl.when(i == 0)
    def _():
        xs_ref[...] = part

    @pl.when(i > 0)
    def _():
        xs_ref[...] = xs_ref[...] + part

    @pl.when(i == _NGRID - 1)
    def _():
        cnt = xs_ref[:, 16:17]
        gidx = lax.broadcasted_iota(jnp.int32, (G, G), 0)
        jidx = lax.broadcasted_iota(jnp.int32, (G, G), 1)
        lt = (jidx < gidx).astype(F32)
        starts = jnp.dot(lt, cnt, precision=lax.Precision.HIGHEST)
        xs_ref[:, 17:18] = starts


_node_call = pl.pallas_call(
    _node_body,
    grid=(_NGRID,),
    in_specs=[
        pl.BlockSpec((_NBLK, D), lambda i: (i, 0)),
        pl.BlockSpec((8, D), lambda i: (0, 0)),
        pl.BlockSpec((1, 1, _NBLK), lambda i: (i, 0, 0)),
        pl.BlockSpec((D, H), lambda i: (0, 0)),
        pl.BlockSpec((1, H), lambda i: (0, 0)),
        pl.BlockSpec((H, H), lambda i: (0, 0)),
        pl.BlockSpec((1, H), lambda i: (0, 0)),
        pl.BlockSpec((H, 128), lambda i: (0, 0)),
        pl.BlockSpec((D, 128), lambda i: (0, 0)),
        pl.BlockSpec((1, 128), lambda i: (0, 0)),
    ],
    out_specs=[
        pl.BlockSpec((_NBLK, 128), lambda i: (i, 0)),
        pl.BlockSpec((G, 32), lambda i: (0, 0)),
    ],
    out_shape=[
        jax.ShapeDtypeStruct((N, 128), F32),
        jax.ShapeDtypeStruct((G, 32), F32),
    ],
)


_NC = 2
_NS = 16
_NW = _NC * _NS
_MAC = 160
_CHS = ((0, 80), (80, 80))


def _gather_body(total_mac, t_hbm, dst_hbm, src_hbm, pp_hbm,
                 dsti_a, srci_a, qd_a, qs_a, pre_a,
                 dsti_b, srci_b, qd_b, qs_b, pre_b, sem_a, sem_b):
    c = lax.axis_index("c")
    s = lax.axis_index("s")
    wid = s * _NC + c
    nw = (total_mac - wid + _NW - 1) // _NW

    def off_of(k):
        return pl.multiple_of((wid + k * _NW) * _MAC, _MAC)

    def start(k, dsti, srci, qd_v, qs_v, pre_v, sem):
        off = off_of(k)
        pltpu.sync_copy(dst_hbm.at[pl.ds(off, _MAC)], dsti)
        pltpu.sync_copy(src_hbm.at[pl.ds(off, _MAC)], srci)
        for o, n in _CHS:
            sl = pl.ds(o, n)
            pltpu.async_copy(t_hbm.at[dsti.at[sl]], qd_v.at[sl], sem)
            pltpu.async_copy(t_hbm.at[srci.at[sl]], qs_v.at[sl], sem)

    def finish(k, dsti, srci, qd_v, qs_v, pre_v, sem):
        off = off_of(k)
        for o, n in _CHS:
            sl = pl.ds(o, n)
            pltpu.make_async_copy(t_hbm.at[dsti.at[sl]], qd_v.at[sl],
                                  sem).wait()
            pltpu.make_async_copy(t_hbm.at[srci.at[sl]], qs_v.at[sl],
                                  sem).wait()

        @functools.partial(plsc.parallel_loop, 0, _MAC, unroll=8)
        def _(e):
            r = e // 2
            cb = (e % 2) * 64
            for l in range(4):
                a = qd_v[e, pl.ds(l * 16, 16)]
                b = qs_v[e, pl.ds(64 + l * 16, 16)]
                pre_v[r, pl.ds(cb + l * 16, 16)] = jnp.maximum(a + b, 0.0)

        row = pl.multiple_of(off // 2, _MAC // 2)
        pltpu.sync_copy(pre_v, pp_hbm.at[pl.ds(row, _MAC // 2)])

    bufs_a = (dsti_a, srci_a, qd_a, qs_a, pre_a, sem_a)
    bufs_b = (dsti_b, srci_b, qd_b, qs_b, pre_b, sem_b)

    start(0, *bufs_a)

    def body(i, carry):
        k = 2 * i
        start(k + 1, *bufs_b)
        finish(k, *bufs_a)

        @pl.when(k + 2 < nw)
        def _():
            start(k + 2, *bufs_a)

        finish(k + 1, *bufs_b)
        return carry

    lax.fori_loop(0, nw // 2, body, 0)

    @pl.when(nw % 2 == 1)
    def _():
        finish(nw - 1, *bufs_a)


@functools.cache
def _make_gather_call(ne):
    total_mac = ne // _MAC
    return pl.kernel(
        functools.partial(_gather_body, total_mac),
        out_type=jax.ShapeDtypeStruct((ne // 2, 128), F32),
        mesh=plsc.VectorSubcoreMesh(
            core_axis_name="c", subcore_axis_name="s",
            num_cores=_NC, num_subcores=_NS),
        scratch_types=[
            pltpu.VMEM((_MAC,), jnp.int32),
            pltpu.VMEM((_MAC,), jnp.int32),
            pltpu.VMEM((_MAC, 128), F32),
            pltpu.VMEM((_MAC, 128), F32),
            pltpu.VMEM((_MAC // 2, 128), F32),
            pltpu.VMEM((_MAC,), jnp.int32),
            pltpu.VMEM((_MAC,), jnp.int32),
            pltpu.VMEM((_MAC, 128), F32),
            pltpu.VMEM((_MAC, 128), F32),
            pltpu.VMEM((_MAC // 2, 128), F32),
            pltpu.SemaphoreType.DMA,
            pltpu.SemaphoreType.DMA,
        ],
    )


_EBLK = 3200


def _edge_body(egrid, pp_ref, de_ref, do_ref, xs_ref, wc2x2_ref, bc2x2_ref,
               acc_ref):
    i = pl.program_id(0)
    h = pp_ref[...].astype(BF16)
    m2 = jnp.tanh(
        jnp.dot(h, wc2x2_ref[...].astype(BF16), preferred_element_type=F32)
        + bc2x2_ref[...])

    cnt = xs_ref[:, 16:17]
    starts = xs_ref[:, 17:18]
    ends = starts + cnt

    de = de_ref[0, 0, :]
    do = do_ref[0, 0, :]
    oh_e = ((de[None, :] >= starts)
            & (de[None, :] < ends)).astype(BF16)
    oh_o = ((do[None, :] >= starts)
            & (do[None, :] < ends)).astype(BF16)
    part = (jnp.dot(oh_e, m2[:, :H].astype(BF16), preferred_element_type=F32)
            + jnp.dot(oh_o, m2[:, H:].astype(BF16),
                      preferred_element_type=F32))

    @pl.when(i == 0)
    def _():
        acc_ref[...] = part

    @pl.when(i > 0)
    def _():
        acc_ref[...] = acc_ref[...] + part


@functools.cache
def _make_edge_call(ne):
    egrid = ne // _EBLK
    eb2 = _EBLK // 2
    return pl.pallas_call(
        functools.partial(_edge_body, egrid),
        grid=(egrid,),
        in_specs=[
            pl.BlockSpec((eb2, 128), lambda i: (i, 0)),
            pl.BlockSpec((1, 1, eb2), lambda i: (i, 0, 0)),
            pl.BlockSpec((1, 1, eb2), lambda i: (i, 0, 0)),
            pl.BlockSpec((G, 32), lambda i: (0, 0)),
            pl.BlockSpec((128, 2 * H), lambda i: (0, 0)),
            pl.BlockSpec((1, 2 * H), lambda i: (0, 0)),
        ],
        out_specs=pl.BlockSpec((G, H), lambda i: (0, 0)),
        out_shape=jax.ShapeDtypeStruct((G, H), F32),
    )


def _final_body(acc1_ref, acc2_ref, xs_ref, wo1_ref, bo1_ref,
                wo2_ref, bo2_ref, out_ref):
    cnt = xs_ref[:, 16:17]
    sums = jnp.concatenate(
        [acc1_ref[...] + acc2_ref[...], xs_ref[:, :D]], axis=1)
    xm = sums / jnp.maximum(cnt, 1.0)
    o1 = jnp.maximum(
        jnp.dot(xm, wo1_ref[...], precision=lax.Precision.HIGHEST)
        + bo1_ref[...], 0.0)
    z = (jnp.dot(o1, wo2_ref[...], precision=lax.Precision.HIGHEST)
         + bo2_ref[...])
    out_ref[...] = 1.0 / (1.0 + jnp.exp(-z))


_final_call = pl.pallas_call(
    _final_body,
    out_shape=jax.ShapeDtypeStruct((G, 1), F32),
)


def kernel(x, edge_index, batch, bn_w, bn_b, W1, b1, W2, b2,
           Wc1, bc1, Wc2, bc2, Wo1, bo1, Wo2, bo2):
    src = edge_index[0]
    dst = edge_index[1]
    e2 = E // 2

    stats = _stats_call(x, bn_w.reshape(1, D), bn_b.reshape(1, D))

    w128 = jnp.concatenate([Wc1[:H + D] - Wc1[H + D:], Wc1[H + D:]], axis=1)
    bt = jnp.concatenate([bc1, jnp.zeros((2 * H,), F32)]).reshape(1, 128)
    batch3 = batch.astype(F32).reshape(_NGRID, 1, _NBLK)
    t, xs = _node_call(x, stats, batch3,
                       W1, b1.reshape(1, H), W2, b2.reshape(1, H),
                       w128[:H], w128[H:], bt)

    gather = _make_gather_call(e2)
    edge = _make_edge_call(e2)
    dstf = dst.astype(F32)
    wc2x2 = jnp.zeros((128, 2 * H), F32)
    wc2x2 = wc2x2.at[:2 * H, :H].set(Wc2).at[2 * H:, H:].set(Wc2)
    bc2x2 = jnp.concatenate([bc2, bc2]).reshape(1, 2 * H)
    nb = e2 // _EBLK
    eb2 = _EBLK // 2

    pp1 = gather(t, dst[:e2], src[:e2])
    pp2 = gather(t, dst[e2:], src[e2:])
    de1 = dstf[0:e2:2].reshape(nb, 1, eb2)
    do1 = dstf[1:e2:2].reshape(nb, 1, eb2)
    de2 = dstf[e2::2].reshape(nb, 1, eb2)
    do2 = dstf[e2 + 1::2].reshape(nb, 1, eb2)
    acc1 = edge(pp1, de1, do1, xs, wc2x2, bc2x2)
    acc2 = edge(pp2, de2, do2, xs, wc2x2, bc2x2)

    out = _final_call(acc1, acc2, xs, Wo1, bo1.reshape(1, H),
                      Wo2, bo2.reshape(1, 1))
    return out
